# Initial kernel scaffold; baseline (speedup 1.0000x reference)
import jax
import jax.numpy as jnp
from jax.experimental import pallas as pl


def kernel(i1, i2, target_table, context_table):
    raise NotImplementedError("write your pallas kernel here")


# speedup vs baseline: 1.0275x; 1.0275x over previous
import functools

import jax
import jax.numpy as jnp
from jax import lax
from jax.experimental import pallas as pl
from jax.experimental.pallas import tpu as pltpu
from jax.experimental.pallas import tpu_sc as plsc

VOCAB = 1000000
D = 32
NEG = 20
B = 16384
L = 16

NC = 2
NS = 16
NW = NC * NS
B_PER_W = B // NW
CHUNK = 64
NCHUNK = B_PER_W // CHUNK
IDX_COLS = 128
ROWS_PER_CHUNK = CHUNK * NEG // IDX_COLS
NEG_PAD = 32


def _body(i1_hbm, i2_hbm, tgt_hbm, ctx_hbm, out_hbm,
          idx1_v, idx2_v, x_v, y_v, zo_v, sem):
    wid = lax.axis_index("s") * NC + lax.axis_index("c")
    neg_pad = jnp.where(lax.iota(jnp.int32, L) < (NEG - L), 0.0, -1e30)

    def chunk_body(c, carry):
        base = wid * B_PER_W + c * CHUNK
        pltpu.sync_copy(i1_hbm.at[pl.ds(base, CHUNK)], idx1_v)
        pltpu.sync_copy(i2_hbm.at[pl.ds(base * NEG, CHUNK * NEG)], idx2_v)
        h1 = pltpu.async_copy(tgt_hbm.at[idx1_v], x_v, sem)
        h2 = [
            pltpu.async_copy(ctx_hbm.at[idx2_v.at[pl.ds(j * IDX_COLS, IDX_COLS)]],
                             y_v.at[pl.ds(j * IDX_COLS, IDX_COLS)], sem)
            for j in range(ROWS_PER_CHUNK)
        ]
        h1.wait()
        for h in h2:
            h.wait()

        lane = lax.iota(jnp.int32, L)

        def b_body(b, carry2):
            x0 = x_v[b, pl.ds(0, L)]
            x1 = x_v[b, pl.ds(L, L)]
            z0 = jnp.zeros((L,), jnp.float32)
            z1 = neg_pad
            for n in range(NEG):
                row = b * NEG + n
                y0 = y_v[row, pl.ds(0, L)]
                y1 = y_v[row, pl.ds(L, L)]
                s = jnp.sum(x0 * y0 + x1 * y1)
                if n < L:
                    z0 = jnp.where(lane == n, s, z0)
                else:
                    z1 = jnp.where(lane == (n - L), s, z1)
            m = jnp.max(jnp.maximum(z0, z1))
            e0 = jnp.exp(z0 - m)
            e1 = jnp.exp(z1 - m)
            tot = jnp.broadcast_to(jnp.sum(e0 + e1), (L,))
            zo_v[b, pl.ds(0, L)] = e0 / tot
            zo_v[b, pl.ds(L, L)] = e1 / tot
            return carry2

        lax.fori_loop(0, CHUNK, b_body, 0)
        pltpu.sync_copy(zo_v, out_hbm.at[pl.ds(base, CHUNK)])
        return carry

    lax.fori_loop(0, NCHUNK, chunk_body, 0)


@functools.partial(jax.jit, static_argnames=("interpret",))
def _run(i1f, i2r, target_table, context_table, interpret=False):
    mesh = plsc.VectorSubcoreMesh(
        core_axis_name="c", subcore_axis_name="s",
        num_cores=NC, num_subcores=NS)
    k = pl.kernel(
        _body,
        out_type=jax.ShapeDtypeStruct((B, NEG_PAD), jnp.float32),
        mesh=mesh,
        scratch_types=[
            pltpu.VMEM((CHUNK,), jnp.int32),
            pltpu.VMEM((CHUNK * NEG,), jnp.int32),
            pltpu.VMEM((CHUNK, D), jnp.float32),
            pltpu.VMEM((CHUNK * NEG, D), jnp.float32),
            pltpu.VMEM((CHUNK, NEG_PAD), jnp.float32),
            pltpu.SemaphoreType.DMA,
        ],
        compiler_params=pltpu.CompilerParams(needs_layout_passes=False,
                                             use_tc_tiling_on_sc=False),
        interpret=interpret,
    )
    return k(i1f, i2r, target_table, context_table)


def kernel(i1, i2, target_table, context_table):
    i1f = i1.reshape(B)
    i2r = i2.reshape(B * NEG)
    out = _run(i1f, i2r, target_table, context_table)
    return out[:, :NEG]


# speedup vs baseline: 2.0381x; 1.9836x over previous
import functools

import jax
import jax.numpy as jnp
from jax import lax
from jax.experimental import pallas as pl
from jax.experimental.pallas import tpu as pltpu
from jax.experimental.pallas import tpu_sc as plsc

VOCAB = 1000000
D = 32
NEG = 20
B = 16384
L = 16

NC = 2
NS = 16
NW = NC * NS
B_PER_W = B // NW
CHUNK = 64
NCHUNK = B_PER_W // CHUNK
IDX_COLS = 128
ROWS_PER_CHUNK = CHUNK * NEG // IDX_COLS
NEG_PAD = 32


def _body(i1_hbm, i2_hbm, tgt_hbm, ctx_hbm, out_hbm,
          idx1_v, idx2_v, x_v, y_v, zo_v, sem):
    wid = lax.axis_index("s") * NC + lax.axis_index("c")
    neg_pad = jnp.where(lax.iota(jnp.int32, L) < (NEG - L), 0.0, -1e30)

    def chunk_body(c, carry):
        base = wid * B_PER_W + c * CHUNK
        pltpu.sync_copy(i1_hbm.at[pl.ds(base, CHUNK)], idx1_v)
        pltpu.sync_copy(i2_hbm.at[pl.ds(base * NEG, CHUNK * NEG)], idx2_v)

        def remap(ref, k):
            v = ref[pl.ds(k * L, L)]
            ref[pl.ds(k * L, L)] = (
                ((v >> 9) << 9) | ((v & 127) << 2) | ((v >> 7) & 3))
            return k

        lax.fori_loop(0, CHUNK // L, lambda k, cc: remap(idx1_v, k) * 0 + cc, 0)
        lax.fori_loop(0, CHUNK * NEG // L,
                      lambda k, cc: remap(idx2_v, k) * 0 + cc, 0)
        h1 = pltpu.async_copy(tgt_hbm.at[idx1_v], x_v, sem)
        h2 = [
            pltpu.async_copy(ctx_hbm.at[idx2_v.at[pl.ds(j * IDX_COLS, IDX_COLS)]],
                             y_v.at[pl.ds(j * IDX_COLS, IDX_COLS)], sem)
            for j in range(ROWS_PER_CHUNK)
        ]
        h1.wait()
        for h in h2:
            h.wait()

        lane = lax.iota(jnp.int32, L)

        def b_body(b, carry2):
            x0 = x_v[b, pl.ds(0, L)]
            x1 = x_v[b, pl.ds(L, L)]
            z0 = jnp.zeros((L,), jnp.float32)
            z1 = neg_pad
            for n in range(NEG):
                row = b * NEG + n
                y0 = y_v[row, pl.ds(0, L)]
                y1 = y_v[row, pl.ds(L, L)]
                s = jnp.sum(x0 * y0 + x1 * y1)
                if n < L:
                    z0 = jnp.where(lane == n, s, z0)
                else:
                    z1 = jnp.where(lane == (n - L), s, z1)
            m = jnp.max(jnp.maximum(z0, z1))
            e0 = jnp.exp(z0 - m)
            e1 = jnp.exp(z1 - m)
            tot = jnp.broadcast_to(jnp.sum(e0 + e1), (L,))
            zo_v[b, pl.ds(0, L)] = e0 / tot
            zo_v[b, pl.ds(L, L)] = e1 / tot
            return carry2

        lax.fori_loop(0, CHUNK, b_body, 0)
        pltpu.sync_copy(zo_v, out_hbm.at[pl.ds(base, CHUNK)])
        return carry

    lax.fori_loop(0, NCHUNK, chunk_body, 0)


TCB = 2048
TC_GRID = (VOCAB + TCB - 1) // TCB
VOCAB_PAD = TC_GRID * TCB


def _transpose_body(a_ref, b_ref, oa_ref, ob_ref):
    for src, dst in ((a_ref, oa_ref), (b_ref, ob_ref)):
        x = src[...]
        for m in range(TCB // 512):
            s = jnp.concatenate(
                [x[:, m * 512 + q * 128:m * 512 + (q + 1) * 128]
                 for q in range(4)],
                axis=0)
            dst[pl.ds(m * 128, 128), :] = jnp.swapaxes(s, 0, 1)


def _relayout(tt_t, ct_t):
    return pl.pallas_call(
        _transpose_body,
        grid=(TC_GRID,),
        in_specs=[pl.BlockSpec((D, TCB), lambda i: (0, i))] * 2,
        out_specs=[pl.BlockSpec((TCB // 4, 128), lambda i: (i, 0))] * 2,
        out_shape=[jax.ShapeDtypeStruct((VOCAB_PAD * D // 128, 128),
                                        jnp.float32)] * 2,
    )(tt_t, ct_t)


@functools.partial(jax.jit, static_argnames=("interpret",))
def _run(i1f, i2r, target_table, context_table, interpret=False):
    mesh = plsc.VectorSubcoreMesh(
        core_axis_name="c", subcore_axis_name="s",
        num_cores=NC, num_subcores=NS)
    k = pl.kernel(
        _body,
        out_type=jax.ShapeDtypeStruct((B, NEG_PAD), jnp.float32),
        mesh=mesh,
        scratch_types=[
            pltpu.VMEM((CHUNK,), jnp.int32),
            pltpu.VMEM((CHUNK * NEG,), jnp.int32),
            pltpu.VMEM((CHUNK, D), jnp.float32),
            pltpu.VMEM((CHUNK * NEG, D), jnp.float32),
            pltpu.VMEM((CHUNK, NEG_PAD), jnp.float32),
            pltpu.SemaphoreType.DMA,
        ],
        compiler_params=pltpu.CompilerParams(needs_layout_passes=False,
                                             use_tc_tiling_on_sc=False),
        interpret=interpret,
    )
    return k(i1f, i2r, target_table, context_table)


def kernel(i1, i2, target_table, context_table):
    i1f = i1.reshape(B)
    i2r = i2.reshape(B * NEG)
    tt_lin, ct_lin = _relayout(target_table.T, context_table.T)
    out = _run(i1f, i2r, tt_lin.reshape(VOCAB_PAD, D),
               ct_lin.reshape(VOCAB_PAD, D))
    return out[:, :NEG]


# speedup vs baseline: 2.6792x; 1.3146x over previous
import functools

import jax
import jax.numpy as jnp
from jax import lax
from jax.experimental import pallas as pl
from jax.experimental.pallas import tpu as pltpu
from jax.experimental.pallas import tpu_sc as plsc

VOCAB = 1000000
D = 32
NEG = 20
B = 16384
L = 16

NC = 2
NS = 16
NW = NC * NS
B_PER_W = B // NW
CHUNK = 64
NCHUNK = B_PER_W // CHUNK
IDX_COLS = 128
ROWS_PER_CHUNK = CHUNK * NEG // IDX_COLS
NEG_PAD = 32


def _body(i1_hbm, i2_hbm, tgt_hbm, ctx_hbm, out_hbm,
          idx1_v, idx2_v, x_v, y_v, zo_v, gsem0, gsem1, osem):
    wid = lax.axis_index("s") * NC + lax.axis_index("c")
    neg_pad = jnp.where(lax.iota(jnp.int32, L) < (NEG - L), 0.0, -1e30)
    lane = lax.iota(jnp.int32, L)
    gsems = (gsem0, gsem1)

    def remap(ref, off, k):
        v = ref[pl.ds(off + k * L, L)]
        ref[pl.ds(off + k * L, L)] = (
            ((v >> 9) << 9) | ((v & 127) << 2) | ((v >> 7) & 3))
        return k

    def stage(c, p):
        base = wid * B_PER_W + c * CHUNK
        pltpu.sync_copy(i1_hbm.at[pl.ds(base, CHUNK)],
                        idx1_v.at[pl.ds(p * CHUNK, CHUNK)])
        pltpu.sync_copy(i2_hbm.at[pl.ds(base * NEG, CHUNK * NEG)],
                        idx2_v.at[pl.ds(p * CHUNK * NEG, CHUNK * NEG)])
        lax.fori_loop(
            0, CHUNK // L,
            lambda k, cc: remap(idx1_v, p * CHUNK, k) * 0 + cc, 0)
        lax.fori_loop(
            0, CHUNK * NEG // L,
            lambda k, cc: remap(idx2_v, p * CHUNK * NEG, k) * 0 + cc, 0)

    def fire(p):
        hs = [pltpu.async_copy(
            tgt_hbm.at[idx1_v.at[pl.ds(p * CHUNK, CHUNK)]],
            x_v.at[pl.ds(p * CHUNK, CHUNK)], gsems[p])]
        hs += [
            pltpu.async_copy(
                ctx_hbm.at[idx2_v.at[pl.ds(p * CHUNK * NEG + j * IDX_COLS,
                                           IDX_COLS)]],
                y_v.at[pl.ds(p * CHUNK * NEG + j * IDX_COLS, IDX_COLS)],
                gsems[p])
            for j in range(ROWS_PER_CHUNK)
        ]
        return hs

    def compute(p):
        xo = p * CHUNK
        yo = p * CHUNK * NEG

        def b_body(b, carry2):
            x0 = x_v[xo + b, pl.ds(0, L)]
            x1 = x_v[xo + b, pl.ds(L, L)]
            z0 = jnp.zeros((L,), jnp.float32)
            z1 = neg_pad
            for n in range(NEG):
                row = yo + b * NEG + n
                y0 = y_v[row, pl.ds(0, L)]
                y1 = y_v[row, pl.ds(L, L)]
                s = jnp.sum(x0 * y0 + x1 * y1)
                if n < L:
                    z0 = jnp.where(lane == n, s, z0)
                else:
                    z1 = jnp.where(lane == (n - L), s, z1)
            m = jnp.max(jnp.maximum(z0, z1))
            e0 = jnp.exp(z0 - m)
            e1 = jnp.exp(z1 - m)
            tot = jnp.broadcast_to(jnp.sum(e0 + e1), (L,))
            zo_v[xo + b, pl.ds(0, L)] = e0 / tot
            zo_v[xo + b, pl.ds(L, L)] = e1 / tot
            return carry2

        lax.fori_loop(0, CHUNK, b_body, 0)

    stage(0, 0)
    gh = {0: fire(0)}
    stage(1, 1)
    oh = {}
    for c in range(NCHUNK):
        p = c % 2
        if c + 1 < NCHUNK:
            gh[c + 1] = fire(1 - p)
        for h in gh.pop(c):
            h.wait()
        if c - 2 in oh:
            oh.pop(c - 2).wait()
        compute(p)
        base = wid * B_PER_W + c * CHUNK
        oh[c] = pltpu.async_copy(zo_v.at[pl.ds(p * CHUNK, CHUNK)],
                                 out_hbm.at[pl.ds(base, CHUNK)], osem)
        if c + 2 < NCHUNK:
            stage(c + 2, p)
    for c in sorted(oh):
        oh.pop(c).wait()


TCB = 4096
TC_GRID = (VOCAB + TCB - 1) // TCB
VOCAB_PAD = TC_GRID * TCB


def _transpose_body(a_ref, b_ref, oa_ref, ob_ref):
    for src, dst in ((a_ref, oa_ref), (b_ref, ob_ref)):
        x = src[...]
        for m in range(TCB // 512):
            s = jnp.concatenate(
                [x[:, m * 512 + q * 128:m * 512 + (q + 1) * 128]
                 for q in range(4)],
                axis=0)
            dst[pl.ds(m * 128, 128), :] = jnp.swapaxes(s, 0, 1)


def _relayout(tt_t, ct_t):
    return pl.pallas_call(
        _transpose_body,
        grid=(TC_GRID,),
        in_specs=[pl.BlockSpec((D, TCB), lambda i: (0, i))] * 2,
        out_specs=[pl.BlockSpec((TCB // 4, 128), lambda i: (i, 0))] * 2,
        out_shape=[jax.ShapeDtypeStruct((VOCAB_PAD * D // 128, 128),
                                        jnp.float32)] * 2,
    )(tt_t, ct_t)


@functools.partial(jax.jit, static_argnames=("interpret",))
def _run(i1f, i2r, target_table, context_table, interpret=False):
    mesh = plsc.VectorSubcoreMesh(
        core_axis_name="c", subcore_axis_name="s",
        num_cores=NC, num_subcores=NS)
    k = pl.kernel(
        _body,
        out_type=jax.ShapeDtypeStruct((B, NEG_PAD), jnp.float32),
        mesh=mesh,
        scratch_types=[
            pltpu.VMEM((2 * CHUNK,), jnp.int32),
            pltpu.VMEM((2 * CHUNK * NEG,), jnp.int32),
            pltpu.VMEM((2 * CHUNK, D), jnp.float32),
            pltpu.VMEM((2 * CHUNK * NEG, D), jnp.float32),
            pltpu.VMEM((2 * CHUNK, NEG_PAD), jnp.float32),
            pltpu.SemaphoreType.DMA,
            pltpu.SemaphoreType.DMA,
            pltpu.SemaphoreType.DMA,
        ],
        compiler_params=pltpu.CompilerParams(needs_layout_passes=False,
                                             use_tc_tiling_on_sc=False),
        interpret=interpret,
    )
    return k(i1f, i2r, target_table, context_table)


def kernel(i1, i2, target_table, context_table):
    i1f = i1.reshape(B)
    i2r = i2.reshape(B * NEG)
    tt_lin, ct_lin = _relayout(target_table.T, context_table.T)
    out = _run(i1f, i2r, tt_lin.reshape(VOCAB_PAD, D),
               ct_lin.reshape(VOCAB_PAD, D))
    return out[:, :NEG]
